# baseline (speedup 1.0000x reference)
import functools

import jax
import jax.numpy as jnp
from jax import lax
from jax.experimental import pallas as pl
from jax.experimental.pallas import tpu as pltpu
from jax.experimental.pallas import tpu_sc as plsc

N = 262144
D = 32
K = 64
L = 16
NC = 2
NS = 16
NW = NC * NS
NSC = 32768
NTC = N - NSC
TCB = 1024
PPW = NSC // NW
CHUNK = 1024
NCHUNK = PPW // CHUNK
G = 4
KT = 4
NKT = K // KT
GSETS = CHUNK // (G * L)
LOSS_SCALE = 0.1


def _make_sc_call():
    mesh = plsc.VectorSubcoreMesh(core_axis_name="c", subcore_axis_name="s")

    @functools.partial(
        pl.kernel,
        mesh=mesh,
        out_type=jax.ShapeDtypeStruct((NW, L), jnp.float32),
        compiler_params=pltpu.CompilerParams(
            needs_layout_passes=False, use_tc_tiling_on_sc=False),
        scratch_types=[
            pltpu.VMEM((D, CHUNK), jnp.float32),
            pltpu.VMEM((K, D, L), jnp.float32),
            pltpu.VMEM((K, L), jnp.float32),
            pltpu.VMEM((K, G, L), jnp.float32),
            pltpu.VMEM((L,), jnp.float32),
            pltpu.VMEM((L,), jnp.float32),
        ],
    )
    def sc_kernel(et_hbm, cb_hbm, a_hbm, out_hbm, ebuf, cbuf, cnbuf, tbuf,
                  abuf, lbuf):
        wid = lax.axis_index("s") * NC + lax.axis_index("c")
        base = wid * PPW

        pltpu.sync_copy(cb_hbm, cbuf)
        pltpu.sync_copy(a_hbm, abuf)
        neg_alpha = -abuf[...]
        zero = jnp.zeros((L,), jnp.float32)

        def cn_body(k, _):
            cn = [zero] * 4
            for d in range(D):
                cv = cbuf[k, d, :]
                cn[d % 4] = cn[d % 4] + cv * cv
            cnbuf[k, :] = (cn[0] + cn[1]) + (cn[2] + cn[3])
            return 0

        lax.fori_loop(0, K, cn_body, 0)

        def chunk_body(c, acc):
            for d in range(D):
                pltpu.sync_copy(
                    et_hbm.at[d, pl.ds(base + c * CHUNK, CHUNK)],
                    ebuf.at[d, :])

            def gs_body(gs, acc):
                goff = [gs * (G * L) + g * L for g in range(G)]

                ep = [[zero] * 4 for _ in range(G)]
                for d in range(D):
                    for g in range(G):
                        ev = ebuf[d, pl.ds(goff[g], L)]
                        ep[g][d % 4] = ep[g][d % 4] + ev * ev
                e2 = [(ep[g][0] + ep[g][1]) + (ep[g][2] + ep[g][3])
                      for g in range(G)]

                def kt_body(kt, m):
                    k0 = kt * KT
                    a = [[zero] * G for _ in range(KT)]
                    for d in range(D):
                        cv = [cbuf[k0 + j, d, :] for j in range(KT)]
                        ev = [ebuf[d, pl.ds(goff[g], L)] for g in range(G)]
                        for j in range(KT):
                            for g in range(G):
                                a[j][g] = a[j][g] + cv[j] * ev[g]
                    m = list(m)
                    for j in range(KT):
                        cn = cnbuf[k0 + j, :]
                        for g in range(G):
                            t = cn + a[j][g] * (-2.0)
                            tbuf[k0 + j, g, :] = t
                            m[g] = jnp.minimum(m[g], t)
                    return tuple(m)

                m = lax.fori_loop(
                    0, NKT, kt_body,
                    tuple(jnp.full((L,), jnp.inf, jnp.float32)
                          for _ in range(G)))

                def p2_body(kk, carry):
                    s, num = carry
                    s, num = list(s), list(num)
                    for dk in range(2):
                        k = kk * 2 + dk
                        for g in range(G):
                            t = tbuf[k, g, :]
                            w = jnp.exp(neg_alpha * (t - m[g]))
                            s[g] = s[g] + w
                            num[g] = num[g] + t * w
                    return tuple(s), tuple(num)

                s, num = lax.fori_loop(
                    0, K // 2, p2_body,
                    (tuple(zero for _ in range(G)),
                     tuple(zero for _ in range(G))))

                for g in range(G):
                    acc = acc + (e2[g] + num[g] / s[g])
                return acc

            return lax.fori_loop(0, GSETS, gs_body, acc)

        acc = lax.fori_loop(0, NCHUNK, chunk_body, zero)

        lbuf[...] = acc
        pltpu.sync_copy(lbuf, out_hbm.at[wid])

    return sc_kernel


_SC_KERNEL = _make_sc_call()


def _tc_body(e_ref, c_ref, a_ref, out_ref, e2_ref):
    i = pl.program_id(0)
    e = e_ref[...]
    c = c_ref[...]
    a = a_ref[0, 0]
    cn = jnp.sum(c * c, axis=1, keepdims=True)
    gm = jax.lax.dot_general(
        c, e, (((1,), (1,)), ((), ())),
        preferred_element_type=jnp.float32)
    t = cn - 2.0 * gm
    mins = jnp.min(t, axis=0, keepdims=True)
    w = jnp.exp(-a * (t - mins))
    s = jnp.sum(w, axis=0, keepdims=True)
    num = jnp.sum(t * w, axis=0, keepdims=True)
    contrib = num / s

    @pl.when(i == 0)
    def _():
        out_ref[...] = jnp.zeros_like(out_ref)
        e2_ref[0, 0] = 0.0

    out_ref[...] += contrib
    e2_ref[0, 0] += jnp.sum(e * e)


_TC_KERNEL = pl.pallas_call(
    _tc_body,
    grid=(NTC // TCB,),
    in_specs=[
        pl.BlockSpec((TCB, D), lambda i: (i + NSC // TCB, 0)),
        pl.BlockSpec((K, D), lambda i: (0, 0)),
        pl.BlockSpec((1, 1), lambda i: (0, 0),
                     memory_space=pltpu.SMEM),
    ],
    out_specs=[
        pl.BlockSpec((1, TCB), lambda i: (0, 0)),
        pl.BlockSpec((1, 1), lambda i: (0, 0),
                     memory_space=pltpu.SMEM),
    ],
    out_shape=[
        jax.ShapeDtypeStruct((1, TCB), jnp.float32),
        jax.ShapeDtypeStruct((1, 1), jnp.float32),
    ],
)


def kernel(embeddings, centroids, alpha):
    emb = embeddings.astype(jnp.float32)
    cen = centroids.astype(jnp.float32)
    et_sc = emb[:NSC].T.reshape(D, NSC)
    cb = jnp.broadcast_to(cen[:, :, None], (K, D, L))
    avec = jnp.full((L,), alpha, dtype=jnp.float32)
    asc = jnp.full((1, 1), alpha, dtype=jnp.float32)
    partials_sc = _SC_KERNEL(et_sc, cb, avec)
    partial_tc, e2_tc = _TC_KERNEL(emb, cen, asc)
    total = jnp.sum(partials_sc) + jnp.sum(partial_tc) + e2_tc[0, 0]
    return total * jnp.float32(LOSS_SCALE / N)


# speedup vs baseline: 1.7128x; 1.7128x over previous
import functools

import jax
import jax.numpy as jnp
from jax import lax
from jax.experimental import pallas as pl
from jax.experimental.pallas import tpu as pltpu
from jax.experimental.pallas import tpu_sc as plsc

N = 262144
D = 32
K = 64
L = 16
NC = 2
NS = 16
NW = NC * NS
NSC = 16384
NTC = N - NSC
TCB = 1024
PPW = NSC // NW
CHUNK = 512
NCHUNK = PPW // CHUNK
G = 4
KT = 4
NKT = K // KT
GSETS = CHUNK // (G * L)
LOSS_SCALE = 0.1


def _make_sc_call():
    mesh = plsc.VectorSubcoreMesh(core_axis_name="c", subcore_axis_name="s")

    @functools.partial(
        pl.kernel,
        mesh=mesh,
        out_type=jax.ShapeDtypeStruct((NW, L), jnp.float32),
        compiler_params=pltpu.CompilerParams(
            needs_layout_passes=False, use_tc_tiling_on_sc=False),
        scratch_types=[
            pltpu.VMEM((D, CHUNK), jnp.float32),
            pltpu.VMEM((K, D, L), jnp.float32),
            pltpu.VMEM((K, L), jnp.float32),
            pltpu.VMEM((K, G, L), jnp.float32),
            pltpu.VMEM((L,), jnp.float32),
            pltpu.VMEM((L,), jnp.float32),
        ],
    )
    def sc_kernel(et_hbm, cb_hbm, a_hbm, out_hbm, ebuf, cbuf, cnbuf, tbuf,
                  abuf, lbuf):
        wid = lax.axis_index("s") * NC + lax.axis_index("c")
        base = wid * PPW

        pltpu.sync_copy(cb_hbm, cbuf)
        pltpu.sync_copy(a_hbm, abuf)
        neg_alpha = -abuf[...]
        zero = jnp.zeros((L,), jnp.float32)

        def cn_body(k, _):
            cn = [zero] * 4
            for d in range(D):
                cv = cbuf[k, d, :]
                cn[d % 4] = cn[d % 4] + cv * cv
            cnbuf[k, :] = (cn[0] + cn[1]) + (cn[2] + cn[3])
            return 0

        lax.fori_loop(0, K, cn_body, 0)

        def chunk_body(c, acc):
            for d in range(D):
                pltpu.sync_copy(
                    et_hbm.at[d, pl.ds(base + c * CHUNK, CHUNK)],
                    ebuf.at[d, :])

            def gs_body(gs, acc):
                goff = [gs * (G * L) + g * L for g in range(G)]

                ep = [[zero] * 4 for _ in range(G)]
                for d in range(D):
                    for g in range(G):
                        ev = ebuf[d, pl.ds(goff[g], L)]
                        ep[g][d % 4] = ep[g][d % 4] + ev * ev
                e2 = [(ep[g][0] + ep[g][1]) + (ep[g][2] + ep[g][3])
                      for g in range(G)]

                def kt_body(kt, m):
                    k0 = kt * KT
                    a = [[zero] * G for _ in range(KT)]
                    for d in range(D):
                        cv = [cbuf[k0 + j, d, :] for j in range(KT)]
                        ev = [ebuf[d, pl.ds(goff[g], L)] for g in range(G)]
                        for j in range(KT):
                            for g in range(G):
                                a[j][g] = a[j][g] + cv[j] * ev[g]
                    m = list(m)
                    for j in range(KT):
                        cn = cnbuf[k0 + j, :]
                        for g in range(G):
                            t = cn + a[j][g] * (-2.0)
                            tbuf[k0 + j, g, :] = t
                            m[g] = jnp.minimum(m[g], t)
                    return tuple(m)

                m = lax.fori_loop(
                    0, NKT, kt_body,
                    tuple(jnp.full((L,), jnp.inf, jnp.float32)
                          for _ in range(G)))

                def p2_body(kk, carry):
                    s, num = carry
                    s, num = list(s), list(num)
                    for dk in range(2):
                        k = kk * 2 + dk
                        for g in range(G):
                            t = tbuf[k, g, :]
                            w = jnp.exp(neg_alpha * (t - m[g]))
                            s[g] = s[g] + w
                            num[g] = num[g] + t * w
                    return tuple(s), tuple(num)

                s, num = lax.fori_loop(
                    0, K // 2, p2_body,
                    (tuple(zero for _ in range(G)),
                     tuple(zero for _ in range(G))))

                for g in range(G):
                    acc = acc + (e2[g] + num[g] / s[g])
                return acc

            return lax.fori_loop(0, GSETS, gs_body, acc)

        acc = lax.fori_loop(0, NCHUNK, chunk_body, zero)

        lbuf[...] = acc
        pltpu.sync_copy(lbuf, out_hbm.at[wid])

    return sc_kernel


_SC_KERNEL = _make_sc_call()


def _tc_body(et_ref, c_ref, a_ref, out_ref):
    i = pl.program_id(0)
    et = et_ref[...]
    c = c_ref[...]
    a = a_ref[0, 0]
    cn = jnp.sum(c * c, axis=1, keepdims=True)
    gm = jax.lax.dot_general(
        c, et, (((1,), (0,)), ((), ())),
        preferred_element_type=jnp.float32)
    t = cn - 2.0 * gm
    mins = jnp.min(t, axis=0, keepdims=True)
    w = jnp.exp(-a * (t - mins))
    s = jnp.sum(w, axis=0, keepdims=True)
    num = jnp.sum(t * w, axis=0, keepdims=True)
    e2 = jnp.sum(et * et, axis=0, keepdims=True)
    contrib = e2 + num / s

    @pl.when(i == 0)
    def _():
        out_ref[...] = jnp.zeros_like(out_ref)

    out_ref[...] += contrib


_TC_KERNEL = pl.pallas_call(
    _tc_body,
    grid=(NTC // TCB,),
    in_specs=[
        pl.BlockSpec((D, TCB), lambda i: (0, i + NSC // TCB)),
        pl.BlockSpec((K, D), lambda i: (0, 0)),
        pl.BlockSpec((1, 1), lambda i: (0, 0),
                     memory_space=pltpu.SMEM),
    ],
    out_specs=pl.BlockSpec((1, TCB), lambda i: (0, 0)),
    out_shape=jax.ShapeDtypeStruct((1, TCB), jnp.float32),
)


def kernel(embeddings, centroids, alpha):
    emb = embeddings.astype(jnp.float32)
    cen = centroids.astype(jnp.float32)
    et = emb.T.reshape(D, N)
    cb = jnp.broadcast_to(cen[:, :, None], (K, D, L))
    avec = jnp.full((L,), alpha, dtype=jnp.float32)
    asc = jnp.full((1, 1), alpha, dtype=jnp.float32)
    et_sc = jax.lax.slice(et, (0, 0), (D, NSC))
    partials_sc = _SC_KERNEL(et_sc, cb, avec)
    partial_tc = _TC_KERNEL(et, cen, asc)
    total = jnp.sum(partials_sc) + jnp.sum(partial_tc)
    return total * jnp.float32(LOSS_SCALE / N)
